# Initial kernel scaffold; baseline (speedup 1.0000x reference)
import jax
import jax.numpy as jnp
from jax.experimental import pallas as pl


def kernel(input, emb, W, b):
    raise NotImplementedError("write your pallas kernel here")


# speedup vs baseline: 1.2632x; 1.2632x over previous
import functools

import jax
import jax.numpy as jnp
from jax import lax
from jax.experimental import pallas as pl
from jax.experimental.pallas import tpu as pltpu
from jax.experimental.pallas import tpu_sc as plsc

N_VOCAB = 4 * 10 ** 6
DIM = 5
LANES = 128
GROUP = DIM * LANES
T_ROWS = N_VOCAB // LANES
BLK = 2048

B_TOTAL = 16384 * 200
IDX_COLS = 128
IDX_ROWS = B_TOTAL // IDX_COLS
NC, NS = 2, 16
NW = NC * NS
ROWS_PER_W = IDX_ROWS // NW
K = 16
N_CHUNKS = ROWS_PER_W // K


def _fuse_body(emb_ref, w_ref, b_ref, out_ref):
    x = emb_ref[...]
    r = lax.broadcasted_iota(jnp.int32, (GROUP, LANES), 0)
    c = lax.broadcasted_iota(jnp.int32, (GROUP, LANES), 1)
    acc = jnp.zeros((GROUP, LANES), jnp.float32)
    for k in range(DIM):
        acc = acc + jnp.where(r == c * DIM + k, w_ref[0, k], 0.0)
    out_ref[...] = (
        jnp.dot(x, acc, preferred_element_type=jnp.float32) + b_ref[0]
    )


def _fuse_table(emb_r, W, b):
    grid = (T_ROWS + BLK - 1) // BLK
    return pl.pallas_call(
        _fuse_body,
        grid=(grid,),
        in_specs=[
            pl.BlockSpec((BLK, GROUP), lambda i: (i, 0)),
            pl.BlockSpec(memory_space=pltpu.SMEM),
            pl.BlockSpec(memory_space=pltpu.SMEM),
        ],
        out_specs=pl.BlockSpec((BLK, LANES), lambda i: (i, 0)),
        out_shape=jax.ShapeDtypeStruct((T_ROWS, LANES), jnp.float32),
    )(emb_r, W, b)


def _gather_body(t_hbm, idx_hbm, out_hbm, idx_v, val_v, sem):
    wid = lax.axis_index("s") * NC + lax.axis_index("c")
    base = wid * ROWS_PER_W

    def chunk(ci, carry):
        row0 = base + ci * K
        pltpu.sync_copy(idx_hbm.at[pl.ds(row0, K)], idx_v)
        copies = [
            pltpu.async_copy(t_hbm.at[idx_v.at[j]], val_v.at[j], sem)
            for j in range(K)
        ]
        for cp in copies:
            cp.wait()
        pltpu.sync_copy(val_v, out_hbm.at[pl.ds(row0, K)])
        return carry

    lax.fori_loop(0, N_CHUNKS, chunk, 0)


@functools.cache
def _gather_kernel():
    return pl.kernel(
        _gather_body,
        mesh=plsc.VectorSubcoreMesh(core_axis_name="c", subcore_axis_name="s"),
        out_type=jax.ShapeDtypeStruct((IDX_ROWS, IDX_COLS), jnp.float32),
        scratch_types=[
            pltpu.VMEM((K, IDX_COLS), jnp.int32),
            pltpu.VMEM((K, IDX_COLS), jnp.float32),
            pltpu.SemaphoreType.DMA,
        ],
    )


def kernel(input, emb, W, b):
    t = _fuse_table(emb.reshape(T_ROWS, GROUP), W, b)
    idx = input.astype(jnp.int32).reshape(IDX_ROWS, IDX_COLS)
    out = _gather_kernel()(t.reshape(N_VOCAB), idx)
    return out.reshape(16384, 200, 1)


# speedup vs baseline: 7.0363x; 5.5703x over previous
import functools

import jax
import jax.numpy as jnp
from jax import lax
from jax.experimental import pallas as pl
from jax.experimental.pallas import tpu as pltpu
from jax.experimental.pallas import tpu_sc as plsc

N_VOCAB = 4 * 10 ** 6
DIM = 5
LANES = 128
T_ROWS = N_VOCAB // LANES
BLKR = 1024
BLKL = BLKR * LANES

B_TOTAL = 16384 * 200
IDX_COLS = 128
IDX_ROWS = B_TOTAL // IDX_COLS
NC, NS = 2, 16
NW = NC * NS
ROWS_PER_W = IDX_ROWS // NW
K = 16
N_CHUNKS = ROWS_PER_W // K


def _fuse_body(embT_ref, w_ref, b_ref, out_ref):
    x = embT_ref[...]
    acc = x[0] * w_ref[0, 0]
    for k in range(1, DIM):
        acc = acc + x[k] * w_ref[0, k]
    out_ref[...] = (acc + b_ref[0]).reshape(BLKR, LANES)


def _fuse_table(embT, W, b):
    grid = (N_VOCAB + BLKL - 1) // BLKL
    return pl.pallas_call(
        _fuse_body,
        grid=(grid,),
        in_specs=[
            pl.BlockSpec((DIM, BLKL), lambda i: (0, i)),
            pl.BlockSpec(memory_space=pltpu.SMEM),
            pl.BlockSpec(memory_space=pltpu.SMEM),
        ],
        out_specs=pl.BlockSpec((BLKR, LANES), lambda i: (i, 0)),
        out_shape=jax.ShapeDtypeStruct((T_ROWS, LANES), jnp.float32),
    )(embT, W, b)


def _gather_body(t_hbm, idx_hbm, out_hbm, idx_v, val_v, sem):
    wid = lax.axis_index("s") * NC + lax.axis_index("c")
    base = wid * ROWS_PER_W

    def chunk(ci, carry):
        row0 = base + ci * K
        pltpu.sync_copy(idx_hbm.at[pl.ds(row0, K)], idx_v)
        copies = [
            pltpu.async_copy(t_hbm.at[idx_v.at[j]], val_v.at[j], sem)
            for j in range(K)
        ]
        for cp in copies:
            cp.wait()
        pltpu.sync_copy(val_v, out_hbm.at[pl.ds(row0, K)])
        return carry

    lax.fori_loop(0, N_CHUNKS, chunk, 0)


@functools.cache
def _gather_kernel():
    return pl.kernel(
        _gather_body,
        mesh=plsc.VectorSubcoreMesh(core_axis_name="c", subcore_axis_name="s"),
        out_type=jax.ShapeDtypeStruct((IDX_ROWS, IDX_COLS), jnp.float32),
        scratch_types=[
            pltpu.VMEM((K, IDX_COLS), jnp.int32),
            pltpu.VMEM((K, IDX_COLS), jnp.float32),
            pltpu.SemaphoreType.DMA,
        ],
    )


def kernel(input, emb, W, b):
    t = _fuse_table(emb.T, W, b)
    idx = input.T.astype(jnp.int32).reshape(IDX_ROWS, IDX_COLS)
    out = _gather_kernel()(t.reshape(N_VOCAB), idx)
    return out.reshape(200, 16384).T.reshape(16384, 200, 1)


# speedup vs baseline: 8.6823x; 1.2339x over previous
import functools

import jax
import jax.numpy as jnp
from jax import lax
from jax.experimental import pallas as pl
from jax.experimental.pallas import tpu as pltpu
from jax.experimental.pallas import tpu_sc as plsc

N_VOCAB = 4 * 10 ** 6
DIM = 5
LANES = 128
T_ROWS = N_VOCAB // LANES
BLKR = 1024
BLKL = BLKR * LANES

B_TOTAL = 16384 * 200
IDX_COLS = 128
IDX_ROWS = B_TOTAL // IDX_COLS
NC, NS = 2, 16
NW = NC * NS
ROWS_PER_W = IDX_ROWS // NW
K = 16
N_CHUNKS = ROWS_PER_W // K


def _fuse_body(embT_ref, w_ref, b_ref, out_ref):
    x = embT_ref[...]
    acc = x[0] * w_ref[0, 0]
    for k in range(1, DIM):
        acc = acc + x[k] * w_ref[0, k]
    out_ref[...] = (acc + b_ref[0]).reshape(BLKR, LANES)


def _fuse_table(embT, W, b):
    grid = (N_VOCAB + BLKL - 1) // BLKL
    return pl.pallas_call(
        _fuse_body,
        grid=(grid,),
        in_specs=[
            pl.BlockSpec((DIM, BLKL), lambda i: (0, i)),
            pl.BlockSpec(memory_space=pltpu.SMEM),
            pl.BlockSpec(memory_space=pltpu.SMEM),
        ],
        out_specs=pl.BlockSpec((BLKR, LANES), lambda i: (i, 0)),
        out_shape=jax.ShapeDtypeStruct((T_ROWS, LANES), jnp.float32),
    )(embT, W, b)


def _gather_body(t_hbm, idx_hbm, out_hbm, idx_v, val_v,
                 isem0, isem1, gsem0, gsem1, osem0, osem1):
    isem = (isem0, isem1)
    gsem = (gsem0, gsem1)
    osem = (osem0, osem1)
    wid = lax.axis_index("s") * NC + lax.axis_index("c")
    base = wid * ROWS_PER_W

    def start_idx(c, b):
        pltpu.async_copy(idx_hbm.at[pl.ds(base + c * K, K)], idx_v.at[b],
                         isem[b])

    def wait_idx(b):
        pltpu.make_async_copy(idx_hbm.at[pl.ds(base, K)], idx_v.at[b],
                              isem[b]).wait()

    def fire(c, b):
        del c
        for j in range(K):
            pltpu.async_copy(t_hbm.at[idx_v.at[b].at[j]], val_v.at[b].at[j],
                             gsem[b])

    def drain(b):
        pltpu.make_async_copy(out_hbm.at[pl.ds(base, K)], val_v.at[b],
                              gsem[b]).wait()

    def start_out(c, b):
        pltpu.async_copy(val_v.at[b], out_hbm.at[pl.ds(base + c * K, K)],
                         osem[b])

    def wait_out(b):
        pltpu.make_async_copy(val_v.at[b], out_hbm.at[pl.ds(base, K)],
                              osem[b]).wait()

    def step(c, b):
        b1 = 1 - b
        wait_out(b1)
        wait_idx(b1)
        fire(c + 1, b1)
        drain(b)
        start_out(c, b)
        start_idx(c + 2, b)

    start_idx(0, 0)
    start_idx(1, 1)
    wait_idx(0)
    fire(0, 0)
    wait_idx(1)
    fire(1, 1)
    drain(0)
    start_out(0, 0)
    start_idx(2, 0)

    def pair(g, carry):
        step(2 * g + 1, 1)
        step(2 * g + 2, 0)
        return carry

    lax.fori_loop(0, (N_CHUNKS - 4) // 2, pair, 0)

    c = N_CHUNKS - 3
    wait_out(0)
    wait_idx(0)
    fire(c + 1, 0)
    drain(1)
    start_out(c, 1)
    start_idx(c + 2, 1)
    c = N_CHUNKS - 2
    wait_out(1)
    wait_idx(1)
    fire(c + 1, 1)
    drain(0)
    start_out(c, 0)
    c = N_CHUNKS - 1
    drain(1)
    start_out(c, 1)
    wait_out(0)
    wait_out(1)


@functools.cache
def _gather_kernel():
    return pl.kernel(
        _gather_body,
        mesh=plsc.VectorSubcoreMesh(core_axis_name="c", subcore_axis_name="s"),
        out_type=jax.ShapeDtypeStruct((IDX_ROWS, IDX_COLS), jnp.float32),
        scratch_types=[
            pltpu.VMEM((2, K, IDX_COLS), jnp.int32),
            pltpu.VMEM((2, K, IDX_COLS), jnp.float32),
            pltpu.SemaphoreType.DMA,
            pltpu.SemaphoreType.DMA,
            pltpu.SemaphoreType.DMA,
            pltpu.SemaphoreType.DMA,
            pltpu.SemaphoreType.DMA,
            pltpu.SemaphoreType.DMA,
        ],
    )


def kernel(input, emb, W, b):
    t = _fuse_table(emb.T, W, b)
    idx = input.T.astype(jnp.int32).reshape(IDX_ROWS, IDX_COLS)
    out = _gather_kernel()(t.reshape(N_VOCAB), idx)
    return out.reshape(200, 16384).T.reshape(16384, 200, 1)


# speedup vs baseline: 9.4368x; 1.0869x over previous
import functools

import jax
import jax.numpy as jnp
from jax import lax
from jax.experimental import pallas as pl
from jax.experimental.pallas import tpu as pltpu
from jax.experimental.pallas import tpu_sc as plsc

N_VOCAB = 4 * 10 ** 6
DIM = 5
LANES = 128
T_ROWS = N_VOCAB // LANES
BLKR = 1024
BLKL = BLKR * LANES

B_TOTAL = 16384 * 200
IDX_COLS = 128
IDX_ROWS = B_TOTAL // IDX_COLS
NC, NS = 2, 16
NW = NC * NS
ROWS_PER_W = IDX_ROWS // NW
K = 16
N_CHUNKS = ROWS_PER_W // K


def _fuse_body(embT_ref, wcol_ref, b_ref, out_ref):
    x = embT_ref[...]
    xr = x.reshape(DIM * BLKR, LANES)
    acc = xr[0:BLKR] * wcol_ref[0, 0]
    for k in range(1, DIM):
        acc = acc + xr[k * BLKR:(k + 1) * BLKR] * wcol_ref[k, 0]
    out_ref[...] = acc + b_ref[0]


def _fuse_table(embT, W, b):
    grid = (N_VOCAB + BLKL - 1) // BLKL
    return pl.pallas_call(
        _fuse_body,
        grid=(grid,),
        in_specs=[
            pl.BlockSpec((DIM, BLKL), lambda i: (0, i)),
            pl.BlockSpec(memory_space=pltpu.SMEM),
            pl.BlockSpec(memory_space=pltpu.SMEM),
        ],
        out_specs=pl.BlockSpec((BLKR, LANES), lambda i: (i, 0)),
        out_shape=jax.ShapeDtypeStruct((T_ROWS, LANES), jnp.float32),
    )(embT, W.T, b)


def _gather_body(t_hbm, idx_hbm, out_hbm, idx_v, val_v,
                 isem0, isem1, gsem0, gsem1, osem0, osem1):
    isem = (isem0, isem1)
    gsem = (gsem0, gsem1)
    osem = (osem0, osem1)
    wid = lax.axis_index("s") * NC + lax.axis_index("c")
    base = wid * ROWS_PER_W

    def start_idx(c, b):
        pltpu.async_copy(idx_hbm.at[pl.ds(base + c * K, K)], idx_v.at[b],
                         isem[b])

    def wait_idx(b):
        pltpu.make_async_copy(idx_hbm.at[pl.ds(base, K)], idx_v.at[b],
                              isem[b]).wait()

    def fire(c, b):
        del c
        for j in range(K):
            pltpu.async_copy(t_hbm.at[idx_v.at[b].at[j]], val_v.at[b].at[j],
                             gsem[b])

    def drain(b):
        pltpu.make_async_copy(out_hbm.at[pl.ds(base, K)], val_v.at[b],
                              gsem[b]).wait()

    def start_out(c, b):
        pltpu.async_copy(val_v.at[b], out_hbm.at[pl.ds(base + c * K, K)],
                         osem[b])

    def wait_out(b):
        pltpu.make_async_copy(val_v.at[b], out_hbm.at[pl.ds(base, K)],
                              osem[b]).wait()

    def step(c, b):
        b1 = 1 - b
        wait_out(b1)
        wait_idx(b1)
        fire(c + 1, b1)
        drain(b)
        start_out(c, b)
        start_idx(c + 2, b)

    start_idx(0, 0)
    start_idx(1, 1)
    wait_idx(0)
    fire(0, 0)
    wait_idx(1)
    fire(1, 1)
    drain(0)
    start_out(0, 0)
    start_idx(2, 0)

    def pair(g, carry):
        step(2 * g + 1, 1)
        step(2 * g + 2, 0)
        return carry

    lax.fori_loop(0, (N_CHUNKS - 4) // 2, pair, 0)

    c = N_CHUNKS - 3
    wait_out(0)
    wait_idx(0)
    fire(c + 1, 0)
    drain(1)
    start_out(c, 1)
    start_idx(c + 2, 1)
    c = N_CHUNKS - 2
    wait_out(1)
    wait_idx(1)
    fire(c + 1, 1)
    drain(0)
    start_out(c, 0)
    c = N_CHUNKS - 1
    drain(1)
    start_out(c, 1)
    wait_out(0)
    wait_out(1)


@functools.cache
def _gather_kernel():
    return pl.kernel(
        _gather_body,
        mesh=plsc.VectorSubcoreMesh(core_axis_name="c", subcore_axis_name="s"),
        out_type=jax.ShapeDtypeStruct((IDX_ROWS, IDX_COLS), jnp.float32),
        scratch_types=[
            pltpu.VMEM((2, K, IDX_COLS), jnp.int32),
            pltpu.VMEM((2, K, IDX_COLS), jnp.float32),
            pltpu.SemaphoreType.DMA,
            pltpu.SemaphoreType.DMA,
            pltpu.SemaphoreType.DMA,
            pltpu.SemaphoreType.DMA,
            pltpu.SemaphoreType.DMA,
            pltpu.SemaphoreType.DMA,
        ],
    )


def kernel(input, emb, W, b):
    t = _fuse_table(emb.T, W, b)
    idx = input.T.astype(jnp.int32).reshape(IDX_ROWS, IDX_COLS)
    out = _gather_kernel()(t.reshape(N_VOCAB), idx)
    return out.reshape(200, 16384).T.reshape(16384, 200, 1)


# speedup vs baseline: 10.1197x; 1.0724x over previous
import functools

import jax
import jax.numpy as jnp
from jax import lax
from jax.experimental import pallas as pl
from jax.experimental.pallas import tpu as pltpu
from jax.experimental.pallas import tpu_sc as plsc

N_VOCAB = 4 * 10 ** 6
DIM = 5
LANES = 128
T_ROWS = N_VOCAB // LANES
BLKR = 1024
BLKL = BLKR * LANES

B_TOTAL = 16384 * 200
IDX_COLS = 128
NC, NS = 2, 16
NW = NC * NS
K = 16
CHUNK = K * IDX_COLS
N_CHUNKS = B_TOTAL // (NW * CHUNK)
SUBS = 16384 // CHUNK


def _fuse_body(embT_ref, wcol_ref, b_ref, out_ref):
    x = embT_ref[...]
    xr = x.reshape(DIM * BLKR, LANES)
    acc = xr[0:BLKR] * wcol_ref[0, 0]
    for k in range(1, DIM):
        acc = acc + xr[k * BLKR:(k + 1) * BLKR] * wcol_ref[k, 0]
    out_ref[...] = acc + b_ref[0]


def _fuse_table(embT, W, b):
    grid = (N_VOCAB + BLKL - 1) // BLKL
    return pl.pallas_call(
        _fuse_body,
        grid=(grid,),
        in_specs=[
            pl.BlockSpec((DIM, BLKL), lambda i: (0, i)),
            pl.BlockSpec(memory_space=pltpu.SMEM),
            pl.BlockSpec(memory_space=pltpu.SMEM),
        ],
        out_specs=pl.BlockSpec((BLKR, LANES), lambda i: (i, 0)),
        out_shape=jax.ShapeDtypeStruct((T_ROWS, LANES), jnp.float32),
    )(embT, W.T, b)


def _gather_body(t_hbm, idx_hbm, out_hbm, idx_v, val_v,
                 isem0, isem1, gsem0, gsem1, osem0, osem1):
    isem = (isem0, isem1)
    gsem = (gsem0, gsem1)
    osem = (osem0, osem1)
    wid = lax.axis_index("s") * NC + lax.axis_index("c")
    base = wid * N_CHUNKS

    def start_idx(c, b):
        g = base + c
        pltpu.async_copy(
            idx_hbm.at[g // SUBS].at[pl.ds((g % SUBS) * CHUNK, CHUNK)],
            idx_v.at[b], isem[b])

    def wait_idx(b):
        pltpu.make_async_copy(idx_hbm.at[0].at[pl.ds(0, CHUNK)],
                              idx_v.at[b], isem[b]).wait()

    def fire(c, b):
        del c
        for j in range(K):
            sl = pl.ds(j * IDX_COLS, IDX_COLS)
            pltpu.async_copy(t_hbm.at[idx_v.at[b].at[sl]],
                             val_v.at[b].at[sl], gsem[b])

    def drain(b):
        pltpu.make_async_copy(out_hbm.at[pl.ds(0, CHUNK)], val_v.at[b],
                              gsem[b]).wait()

    def start_out(c, b):
        pltpu.async_copy(val_v.at[b],
                         out_hbm.at[pl.ds((base + c) * CHUNK, CHUNK)],
                         osem[b])

    def wait_out(b):
        pltpu.make_async_copy(val_v.at[b], out_hbm.at[pl.ds(0, CHUNK)],
                              osem[b]).wait()

    def step(c, b):
        b1 = 1 - b
        wait_out(b1)
        wait_idx(b1)
        fire(c + 1, b1)
        drain(b)
        start_out(c, b)
        start_idx(c + 2, b)

    start_idx(0, 0)
    start_idx(1, 1)
    wait_idx(0)
    fire(0, 0)
    wait_idx(1)
    fire(1, 1)
    drain(0)
    start_out(0, 0)
    start_idx(2, 0)

    def pair(g, carry):
        step(2 * g + 1, 1)
        step(2 * g + 2, 0)
        return carry

    lax.fori_loop(0, (N_CHUNKS - 4) // 2, pair, 0)

    c = N_CHUNKS - 3
    wait_out(0)
    wait_idx(0)
    fire(c + 1, 0)
    drain(1)
    start_out(c, 1)
    start_idx(c + 2, 1)
    c = N_CHUNKS - 2
    wait_out(1)
    wait_idx(1)
    fire(c + 1, 1)
    drain(0)
    start_out(c, 0)
    c = N_CHUNKS - 1
    drain(1)
    start_out(c, 1)
    wait_out(0)
    wait_out(1)


@functools.cache
def _gather_kernel():
    return pl.kernel(
        _gather_body,
        mesh=plsc.VectorSubcoreMesh(core_axis_name="c", subcore_axis_name="s"),
        out_type=jax.ShapeDtypeStruct((B_TOTAL,), jnp.float32),
        scratch_types=[
            pltpu.VMEM((2, CHUNK), jnp.int32),
            pltpu.VMEM((2, CHUNK), jnp.float32),
            pltpu.SemaphoreType.DMA,
            pltpu.SemaphoreType.DMA,
            pltpu.SemaphoreType.DMA,
            pltpu.SemaphoreType.DMA,
            pltpu.SemaphoreType.DMA,
            pltpu.SemaphoreType.DMA,
        ],
    )


def kernel(input, emb, W, b):
    t = _fuse_table(emb.T, W, b)
    idx = input.T.astype(jnp.int32)
    out = _gather_kernel()(t.reshape(N_VOCAB), idx)
    return out.reshape(200, 16384).T.reshape(16384, 200, 1)


# speedup vs baseline: 10.7902x; 1.0663x over previous
import functools

import jax
import jax.numpy as jnp
from jax import lax
from jax.experimental import pallas as pl
from jax.experimental.pallas import tpu as pltpu
from jax.experimental.pallas import tpu_sc as plsc

N_VOCAB = 4 * 10 ** 6
DIM = 5
LANES = 128
T_ROWS = N_VOCAB // LANES
BLKR = 2048
BLKL = BLKR * LANES

B_TOTAL = 16384 * 200
IDX_COLS = 128
NC, NS = 2, 16
NW = NC * NS
K = 16
CHUNK = K * IDX_COLS
N_CHUNKS = B_TOTAL // (NW * CHUNK)
SUBS = 16384 // CHUNK


def _fuse_body(embT_ref, wcol_ref, b_ref, out_ref):
    x = embT_ref[...]
    xr = x.reshape(DIM * BLKR, LANES)
    acc = xr[0:BLKR] * wcol_ref[0, 0]
    for k in range(1, DIM):
        acc = acc + xr[k * BLKR:(k + 1) * BLKR] * wcol_ref[k, 0]
    out_ref[...] = acc + b_ref[0]


def _fuse_table(embT, W, b):
    grid = (N_VOCAB + BLKL - 1) // BLKL
    return pl.pallas_call(
        _fuse_body,
        grid=(grid,),
        in_specs=[
            pl.BlockSpec((DIM, BLKL), lambda i: (0, i)),
            pl.BlockSpec(memory_space=pltpu.SMEM),
            pl.BlockSpec(memory_space=pltpu.SMEM),
        ],
        out_specs=pl.BlockSpec((BLKR, LANES), lambda i: (i, 0)),
        out_shape=jax.ShapeDtypeStruct((T_ROWS, LANES), jnp.float32),
    )(embT, W.T, b)


def _gather_body(t_hbm, idx_hbm, out_hbm, idx_v, val_v,
                 isem0, isem1, gsem0, gsem1, osem0, osem1):
    isem = (isem0, isem1)
    gsem = (gsem0, gsem1)
    osem = (osem0, osem1)
    wid = lax.axis_index("s") * NC + lax.axis_index("c")
    base = wid * N_CHUNKS

    def start_idx(c, b):
        g = base + c
        pltpu.async_copy(
            idx_hbm.at[g // SUBS].at[pl.ds((g % SUBS) * CHUNK, CHUNK)],
            idx_v.at[b], isem[b])

    def wait_idx(b):
        pltpu.make_async_copy(idx_hbm.at[0].at[pl.ds(0, CHUNK)],
                              idx_v.at[b], isem[b]).wait()

    def fire(c, b):
        del c
        for j in range(K):
            sl = pl.ds(j * IDX_COLS, IDX_COLS)
            pltpu.async_copy(t_hbm.at[idx_v.at[b].at[sl]],
                             val_v.at[b].at[sl], gsem[b])

    def drain(b):
        pltpu.make_async_copy(out_hbm.at[0].at[pl.ds(0, CHUNK)], val_v.at[b],
                              gsem[b]).wait()

    def start_out(c, b):
        g = base + c
        pltpu.async_copy(val_v.at[b],
                         out_hbm.at[g // SUBS].at[pl.ds((g % SUBS) * CHUNK,
                                                        CHUNK)],
                         osem[b])

    def wait_out(b):
        pltpu.make_async_copy(val_v.at[b], out_hbm.at[0].at[pl.ds(0, CHUNK)],
                              osem[b]).wait()

    def step(c, b):
        b1 = 1 - b
        wait_out(b1)
        wait_idx(b1)
        fire(c + 1, b1)
        drain(b)
        start_out(c, b)
        start_idx(c + 2, b)

    start_idx(0, 0)
    start_idx(1, 1)
    wait_idx(0)
    fire(0, 0)
    wait_idx(1)
    fire(1, 1)
    drain(0)
    start_out(0, 0)
    start_idx(2, 0)

    def pair(g, carry):
        step(2 * g + 1, 1)
        step(2 * g + 2, 0)
        return carry

    lax.fori_loop(0, (N_CHUNKS - 4) // 2, pair, 0)

    c = N_CHUNKS - 3
    wait_out(0)
    wait_idx(0)
    fire(c + 1, 0)
    drain(1)
    start_out(c, 1)
    start_idx(c + 2, 1)
    c = N_CHUNKS - 2
    wait_out(1)
    wait_idx(1)
    fire(c + 1, 1)
    drain(0)
    start_out(c, 0)
    c = N_CHUNKS - 1
    drain(1)
    start_out(c, 1)
    wait_out(0)
    wait_out(1)


@functools.cache
def _gather_kernel():
    return pl.kernel(
        _gather_body,
        mesh=plsc.VectorSubcoreMesh(core_axis_name="c", subcore_axis_name="s"),
        out_type=jax.ShapeDtypeStruct((200, 16384), jnp.float32),
        scratch_types=[
            pltpu.VMEM((2, CHUNK), jnp.int32),
            pltpu.VMEM((2, CHUNK), jnp.float32),
            pltpu.SemaphoreType.DMA,
            pltpu.SemaphoreType.DMA,
            pltpu.SemaphoreType.DMA,
            pltpu.SemaphoreType.DMA,
            pltpu.SemaphoreType.DMA,
            pltpu.SemaphoreType.DMA,
        ],
    )


def kernel(input, emb, W, b):
    t = _fuse_table(emb.T, W, b)
    idx = input.T.astype(jnp.int32)
    out = _gather_kernel()(t.reshape(N_VOCAB), idx)
    return out.T.reshape(16384, 200, 1)


# speedup vs baseline: 11.7589x; 1.0898x over previous
import functools

import jax
import jax.numpy as jnp
from jax import lax
from jax.experimental import pallas as pl
from jax.experimental.pallas import tpu as pltpu
from jax.experimental.pallas import tpu_sc as plsc

N_VOCAB = 4 * 10 ** 6
DIM = 5
LANES = 128
T_ROWS = N_VOCAB // LANES
BLKR = 2048
BLKL = BLKR * LANES

B_TOTAL = 16384 * 200
IDX_COLS = 128
NC, NS = 2, 16
NW = NC * NS
K = 32
CHUNK = K * IDX_COLS
N_CHUNKS = B_TOTAL // (NW * CHUNK)
SUBS = 16384 // CHUNK


def _fuse_body(embT_ref, wcol_ref, b_ref, out_ref):
    x = embT_ref[...]
    xr = x.reshape(DIM * BLKR, LANES)
    acc = xr[0:BLKR] * wcol_ref[0, 0]
    for k in range(1, DIM):
        acc = acc + xr[k * BLKR:(k + 1) * BLKR] * wcol_ref[k, 0]
    out_ref[...] = acc + b_ref[0]


def _fuse_table(embT, W, b):
    grid = (N_VOCAB + BLKL - 1) // BLKL
    return pl.pallas_call(
        _fuse_body,
        grid=(grid,),
        in_specs=[
            pl.BlockSpec((DIM, BLKL), lambda i: (0, i)),
            pl.BlockSpec(memory_space=pltpu.SMEM),
            pl.BlockSpec(memory_space=pltpu.SMEM),
        ],
        out_specs=pl.BlockSpec((BLKR, LANES), lambda i: (i, 0)),
        out_shape=jax.ShapeDtypeStruct((T_ROWS, LANES), jnp.float32),
    )(embT, W.T, b)


def _gather_body(t_hbm, idx_hbm, out_hbm, idx_v, val_v,
                 isem0, isem1, gsem0, gsem1, osem0, osem1):
    isem = (isem0, isem1)
    gsem = (gsem0, gsem1)
    osem = (osem0, osem1)
    wid = lax.axis_index("s") * NC + lax.axis_index("c")
    base = wid * N_CHUNKS

    def start_idx(c, b):
        g = base + c
        pltpu.async_copy(
            idx_hbm.at[g // SUBS].at[pl.ds((g % SUBS) * CHUNK, CHUNK)],
            idx_v.at[b], isem[b])

    def wait_idx(b):
        pltpu.make_async_copy(idx_hbm.at[0].at[pl.ds(0, CHUNK)],
                              idx_v.at[b], isem[b]).wait()

    def fire(c, b):
        del c
        for j in range(K):
            sl = pl.ds(j * IDX_COLS, IDX_COLS)
            pltpu.async_copy(t_hbm.at[idx_v.at[b].at[sl]],
                             val_v.at[b].at[sl], gsem[b])

    def drain(b):
        pltpu.make_async_copy(out_hbm.at[0].at[pl.ds(0, CHUNK)], val_v.at[b],
                              gsem[b]).wait()

    def start_out(c, b):
        g = base + c
        pltpu.async_copy(val_v.at[b],
                         out_hbm.at[g // SUBS].at[pl.ds((g % SUBS) * CHUNK,
                                                        CHUNK)],
                         osem[b])

    def wait_out(b):
        pltpu.make_async_copy(val_v.at[b], out_hbm.at[0].at[pl.ds(0, CHUNK)],
                              osem[b]).wait()

    def step(c, b):
        b1 = 1 - b
        wait_out(b1)
        wait_idx(b1)
        fire(c + 1, b1)
        drain(b)
        start_out(c, b)
        start_idx(c + 2, b)

    start_idx(0, 0)
    start_idx(1, 1)
    wait_idx(0)
    fire(0, 0)
    wait_idx(1)
    fire(1, 1)
    drain(0)
    start_out(0, 0)
    start_idx(2, 0)

    def pair(g, carry):
        step(2 * g + 1, 1)
        step(2 * g + 2, 0)
        return carry

    lax.fori_loop(0, (N_CHUNKS - 4) // 2, pair, 0)

    c = N_CHUNKS - 3
    wait_out(0)
    wait_idx(0)
    fire(c + 1, 0)
    drain(1)
    start_out(c, 1)
    start_idx(c + 2, 1)
    c = N_CHUNKS - 2
    wait_out(1)
    wait_idx(1)
    fire(c + 1, 1)
    drain(0)
    start_out(c, 0)
    c = N_CHUNKS - 1
    drain(1)
    start_out(c, 1)
    wait_out(0)
    wait_out(1)


@functools.cache
def _gather_kernel():
    return pl.kernel(
        _gather_body,
        mesh=plsc.VectorSubcoreMesh(core_axis_name="c", subcore_axis_name="s"),
        out_type=jax.ShapeDtypeStruct((200, 16384), jnp.float32),
        scratch_types=[
            pltpu.VMEM((2, CHUNK), jnp.int32),
            pltpu.VMEM((2, CHUNK), jnp.float32),
            pltpu.SemaphoreType.DMA,
            pltpu.SemaphoreType.DMA,
            pltpu.SemaphoreType.DMA,
            pltpu.SemaphoreType.DMA,
            pltpu.SemaphoreType.DMA,
            pltpu.SemaphoreType.DMA,
        ],
    )


def kernel(input, emb, W, b):
    t = _fuse_table(emb.T, W, b)
    idx = input.T.astype(jnp.int32)
    out = _gather_kernel()(t.reshape(N_VOCAB), idx)
    return out.T.reshape(16384, 200, 1)
